# Initial kernel scaffold; baseline (speedup 1.0000x reference)
import jax
import jax.numpy as jnp
from jax.experimental import pallas as pl


def kernel(params, objs, triples):
    raise NotImplementedError("write your pallas kernel here")


# speedup vs baseline: 1.6715x; 1.6715x over previous
import functools

import jax
import jax.numpy as jnp
from jax import lax
from jax.experimental import pallas as pl
from jax.experimental.pallas import tpu as pltpu
from jax.experimental.pallas import tpu_sc as plsc

F32 = jnp.float32

N_NODES = 10000
N_EDGES = 160000
D = 64

NC = 2
NS = 16
NW = NC * NS

NP = 10240
EP = 163840
CH = 128
E_CHUNKS_PER_TILE = EP // (NW * CH)
E_ROWS = EP // CH
NCH = 40
N_ROWS = NP // NCH
PAD_NODE = N_NODES


def _mesh():
    return plsc.VectorSubcoreMesh(core_axis_name="c", subcore_axis_name="s",
                                  num_cores=NC, num_subcores=NS)


_SC_PARAMS = pltpu.CompilerParams(use_tc_tiling_on_sc=False)


@functools.lru_cache(maxsize=None)
def _make_gather(table_rows, n_idx, ch):
    chunks_per_tile = n_idx // (NW * ch)

    @functools.partial(
        pl.kernel,
        out_type=jax.ShapeDtypeStruct((n_idx, D), F32),
        mesh=_mesh(),
        scratch_types=[
            pltpu.VMEM((chunks_per_tile, ch), jnp.int32),
            pltpu.VMEM((ch, D), F32),
        ],
        compiler_params=_SC_PARAMS,
    )
    def gather(tbl_hbm, idx_hbm, out_hbm, idx_v, rows_v):
        wid = lax.axis_index("c") * NS + lax.axis_index("s")
        row0 = wid * chunks_per_tile
        pltpu.sync_copy(idx_hbm.at[pl.ds(row0, chunks_per_tile)], idx_v)

        def body(j, _):
            pltpu.sync_copy(tbl_hbm.at[idx_v.at[j]], rows_v)
            pltpu.sync_copy(rows_v, out_hbm.at[pl.ds((row0 + j) * ch, ch)])
            return 0

        lax.fori_loop(0, chunks_per_tile, body, 0)

    return gather


@functools.lru_cache(maxsize=None)
def _make_scatter2():
    cpt = E_CHUNKS_PER_TILE
    npt = NP // NS

    @functools.partial(
        pl.kernel,
        out_type=jax.ShapeDtypeStruct((NC, NP, D), F32),
        mesh=_mesh(),
        scratch_types=[
            pltpu.VMEM((cpt, CH), jnp.int32),
            pltpu.VMEM((CH, D), F32),
            pltpu.VMEM_SHARED((NP, D), F32),
        ],
        compiler_params=_SC_PARAMS,
    )
    def scatter2(zeros_hbm, s_hbm, vs_hbm, o_hbm, vo_hbm, out_hbm,
                 idx_v, vals_v, acc):
        cid = lax.axis_index("c")
        sid = lax.axis_index("s")
        pltpu.sync_copy(zeros_hbm.at[pl.ds(sid * npt, npt)],
                        acc.at[pl.ds(sid * npt, npt)])
        plsc.subcore_barrier()

        row0 = (cid * NS + sid) * cpt

        def scat(idx_hbm, v_hbm):
            pltpu.sync_copy(idx_hbm.at[pl.ds(row0, cpt)], idx_v)

            def body(j, _):
                pltpu.sync_copy(v_hbm.at[pl.ds((row0 + j) * CH, CH)], vals_v)
                pltpu.sync_copy(vals_v, acc.at[idx_v.at[j]], add=True)
                return 0

            lax.fori_loop(0, cpt, body, 0)

        scat(s_hbm, vs_hbm)
        scat(o_hbm, vo_hbm)
        plsc.subcore_barrier()
        pltpu.sync_copy(acc.at[pl.ds(sid * npt, npt)],
                        out_hbm.at[cid, pl.ds(sid * npt, npt)])

    return scatter2


@functools.lru_cache(maxsize=None)
def _make_count_scatter():
    cpt = E_CHUNKS_PER_TILE
    npt = NP // NS

    @functools.partial(
        pl.kernel,
        out_type=jax.ShapeDtypeStruct((NC, NP, D), F32),
        mesh=_mesh(),
        scratch_types=[
            pltpu.VMEM((cpt, CH), jnp.int32),
            pltpu.VMEM((CH, D), F32),
            pltpu.VMEM_SHARED((NP, D), F32),
        ],
        compiler_params=_SC_PARAMS,
    )
    def count_scatter(zeros_hbm, ones_hbm, s_hbm, o_hbm, out_hbm,
                      idx_v, ones_v, acc):
        cid = lax.axis_index("c")
        sid = lax.axis_index("s")
        pltpu.sync_copy(zeros_hbm.at[pl.ds(sid * npt, npt)],
                        acc.at[pl.ds(sid * npt, npt)])
        pltpu.sync_copy(ones_hbm, ones_v)
        plsc.subcore_barrier()

        row0 = (cid * NS + sid) * cpt

        def scat(idx_hbm):
            pltpu.sync_copy(idx_hbm.at[pl.ds(row0, cpt)], idx_v)

            def body(j, _):
                pltpu.sync_copy(ones_v, acc.at[idx_v.at[j]], add=True)
                return 0

            lax.fori_loop(0, cpt, body, 0)

        scat(s_hbm)
        scat(o_hbm)
        plsc.subcore_barrier()
        pltpu.sync_copy(acc.at[pl.ds(sid * npt, npt)],
                        out_hbm.at[cid, pl.ds(sid * npt, npt)])

    return count_scatter


def _full(shape):
    return pl.BlockSpec(shape, lambda *_: tuple(0 for _ in shape))


def _prep_body(emb_ref, pemb_ref, w1_ref, b1_ref, ta_ref, tb_ref, tp_ref):
    w1 = w1_ref[...]
    emb = emb_ref[...]
    ta_ref[...] = jnp.dot(emb, w1[0:D], preferred_element_type=F32) + b1_ref[...]
    tb_ref[...] = jnp.dot(emb, w1[2 * D:3 * D], preferred_element_type=F32)
    tp_ref[...] = jnp.dot(pemb_ref[...], w1[D:2 * D], preferred_element_type=F32)


def _prep_call(emb, pemb, w1, b1):
    return pl.pallas_call(
        _prep_body,
        out_shape=[
            jax.ShapeDtypeStruct((emb.shape[0], D), F32),
            jax.ShapeDtypeStruct((emb.shape[0], D), F32),
            jax.ShapeDtypeStruct((pemb.shape[0], D), F32),
        ],
    )(emb, pemb, w1, b1)


def _edge_body(gs_ref, go_ref, p_ref, w2_ref, b2_ref, wpn_ref,
               os_ref, oo_ref, pn_ref):
    h = jnp.maximum(gs_ref[...] + go_ref[...] + p_ref[...], 0.0)
    t = jnp.dot(h, w2_ref[...], preferred_element_type=F32) + b2_ref[...]
    t = jnp.maximum(t, 0.0)
    os_ref[...] = t[:, 0:D]
    oo_ref[...] = t[:, 2 * D:3 * D]
    pn_ref[...] = jnp.dot(t[:, D:2 * D], wpn_ref[...],
                          preferred_element_type=F32)


def _edge_body_last(gs_ref, go_ref, p_ref, w2_ref, b2_ref, os_ref, oo_ref):
    h = jnp.maximum(gs_ref[...] + go_ref[...] + p_ref[...], 0.0)
    t = jnp.dot(h, w2_ref[...], preferred_element_type=F32) + b2_ref[...]
    t = jnp.maximum(t, 0.0)
    os_ref[...] = t[:, 0:D]
    oo_ref[...] = t[:, 2 * D:3 * D]


_EB = 1024


def _edge_call(gs, go, p, w2, b2, wpn):
    grid = (EP // _EB,)
    eb = pl.BlockSpec((_EB, D), lambda i: (i, 0))
    if wpn is None:
        return pl.pallas_call(
            _edge_body_last,
            grid=grid,
            in_specs=[eb, eb, eb, _full((D, 3 * D)), _full((1, 3 * D))],
            out_specs=[eb, eb],
            out_shape=[jax.ShapeDtypeStruct((EP, D), F32)] * 2,
        )(gs, go, p, w2, b2)
    return pl.pallas_call(
        _edge_body,
        grid=grid,
        in_specs=[eb, eb, eb, _full((D, 3 * D)), _full((1, 3 * D)),
                  _full((D, D))],
        out_specs=[eb, eb, eb],
        out_shape=[jax.ShapeDtypeStruct((EP, D), F32)] * 3,
    )(gs, go, p, w2, b2, wpn)


def _node_body(p2_ref, c2_ref, w2a_ref, b2a_ref, w2b_ref, b2b_ref,
               wsn_ref, b1n_ref, won_ref, a_ref, b_ref):
    pooled = p2_ref[0] + p2_ref[1]
    cnt = jnp.maximum(c2_ref[0] + c2_ref[1], 1.0)
    x = pooled / cnt
    x = jnp.maximum(jnp.dot(x, w2a_ref[...], preferred_element_type=F32)
                    + b2a_ref[...], 0.0)
    obj = jnp.maximum(jnp.dot(x, w2b_ref[...], preferred_element_type=F32)
                      + b2b_ref[...], 0.0)
    a_ref[...] = jnp.dot(obj, wsn_ref[...], preferred_element_type=F32) + b1n_ref[...]
    b_ref[...] = jnp.dot(obj, won_ref[...], preferred_element_type=F32)


def _node_call(p2, c2, w2a, b2a, w2b, b2b, wsn, b1n, won):
    return pl.pallas_call(
        _node_body,
        out_shape=[jax.ShapeDtypeStruct((NP, D), F32)] * 2,
    )(p2, c2, w2a, b2a, w2b, b2b, wsn, b1n, won)


def _node_last_body(p2_ref, c2_ref, w2a_ref, b2a_ref, w2b_ref, b2b_ref,
                    wb1_ref, bb1_ref, wb2_ref, bb2_ref, out_ref):
    pooled = p2_ref[0] + p2_ref[1]
    cnt = jnp.maximum(c2_ref[0] + c2_ref[1], 1.0)
    x = pooled / cnt
    x = jnp.maximum(jnp.dot(x, w2a_ref[...], preferred_element_type=F32)
                    + b2a_ref[...], 0.0)
    obj = jnp.maximum(jnp.dot(x, w2b_ref[...], preferred_element_type=F32)
                      + b2b_ref[...], 0.0)
    y = jnp.maximum(jnp.dot(obj, wb1_ref[...], preferred_element_type=F32)
                    + bb1_ref[...], 0.0)
    out_ref[...] = jnp.maximum(
        jnp.dot(y, wb2_ref[...], preferred_element_type=F32) + bb2_ref[...], 0.0)


def _node_last_call(p2, c2, w2a, b2a, w2b, b2b, wb1, bb1, wb2, bb2):
    return pl.pallas_call(
        _node_last_body,
        out_shape=jax.ShapeDtypeStruct((NP, D), F32),
    )(p2, c2, w2a, b2a, w2b, b2b, wb1, bb1, wb2, bb2)


def kernel(params, objs, triples):
    s = triples[:, 0]
    p = triples[:, 1]
    o = triples[:, 2]
    pad_e = EP - N_EDGES
    s2 = jnp.concatenate([s, jnp.full((pad_e,), PAD_NODE, jnp.int32)]
                         ).reshape(E_ROWS, CH)
    o2 = jnp.concatenate([o, jnp.full((pad_e,), PAD_NODE, jnp.int32)]
                         ).reshape(E_ROWS, CH)
    p2 = jnp.concatenate([p, jnp.zeros((pad_e,), jnp.int32)]
                         ).reshape(E_ROWS, CH)
    objs2 = jnp.concatenate([objs, jnp.zeros((NP - N_NODES,), jnp.int32)]
                            ).reshape(N_ROWS, NCH)

    obj_emb = jnp.pad(params["obj_emb"], ((0, 3), (0, 0)))
    pred_emb = jnp.pad(params["pred_emb"], ((0, 2), (0, 0)))

    gconv = params["gconv"]
    w1 = [g["net1"][0] for g in gconv]
    b1 = [g["net1"][1].reshape(1, D) for g in gconv]
    w2 = [g["net1"][2] for g in gconv]
    b2 = [g["net1"][3].reshape(1, 3 * D) for g in gconv]
    w2a = [g["net2"][0] for g in gconv]
    b2a = [g["net2"][1].reshape(1, D) for g in gconv]
    w2b = [g["net2"][2] for g in gconv]
    b2b = [g["net2"][3].reshape(1, D) for g in gconv]

    box = params["box_net"]
    wb1, bb1 = box[0], box[1].reshape(1, D)
    wb2 = jnp.pad(box[2], ((0, 0), (0, D - 4)))
    bb2 = jnp.pad(box[3], ((0, D - 4),)).reshape(1, D)

    tabA, tabB, tabP = _prep_call(obj_emb, pred_emb, w1[0], b1[0])

    gather_node = _make_gather(104, NP, NCH)
    gather_pred = _make_gather(48, EP, CH)
    gather_edge = _make_gather(NP, EP, CH)
    scatter2 = _make_scatter2()
    count_scatter = _make_count_scatter()

    A = gather_node(tabA, objs2)
    B = gather_node(tabB, objs2)
    P = gather_pred(tabP, p2)

    zeros_np = jnp.zeros((NP, D), F32)
    ones_ch = jnp.ones((CH, D), F32)
    counts2 = count_scatter(zeros_np, ones_ch, s2, o2)

    for li in range(5):
        gs = gather_edge(A, s2)
        go = gather_edge(B, o2)
        if li < 4:
            os_, oo_, P = _edge_call(gs, go, P, w2[li], b2[li],
                                     w1[li + 1][D:2 * D])
        else:
            os_, oo_ = _edge_call(gs, go, P, w2[li], b2[li], None)
        pooled2 = scatter2(zeros_np, s2, os_, o2, oo_)
        if li < 4:
            A, B = _node_call(pooled2, counts2, w2a[li], b2a[li],
                              w2b[li], b2b[li],
                              w1[li + 1][0:D], b1[li + 1],
                              w1[li + 1][2 * D:3 * D])
        else:
            boxes = _node_last_call(pooled2, counts2, w2a[li], b2a[li],
                                    w2b[li], b2b[li], wb1, bb1, wb2, bb2)

    return boxes[:N_NODES, :4]


# speedup vs baseline: 1.8460x; 1.1044x over previous
import functools

import jax
import jax.numpy as jnp
from jax import lax
from jax.experimental import pallas as pl
from jax.experimental.pallas import tpu as pltpu
from jax.experimental.pallas import tpu_sc as plsc

F32 = jnp.float32

N_NODES = 10000
N_EDGES = 160000
D = 64

NC = 2
NS = 16
NW = NC * NS

NP = 10240
EP = 163840
CH = 128
E_CPT = EP // (NW * CH)
E_ROWS = EP // CH
NCH = 40
N_CPT = NP // (NW * NCH)
N_ROWS = NP // NCH
PAD_NODE = N_NODES
KF = 8


def _mesh():
    return plsc.VectorSubcoreMesh(core_axis_name="c", subcore_axis_name="s",
                                  num_cores=NC, num_subcores=NS)


_SC_PARAMS = pltpu.CompilerParams(use_tc_tiling_on_sc=False)


def _pipelined_gather(tbl, idx_v, out, bufs, gsem, wsem, row0, n_chunks, ch):
    n_groups = n_chunks // KF

    def group(g, _):
        gd = []
        for i in range(KF):
            j = g * KF + i
            gd.append(pltpu.async_copy(tbl.at[idx_v.at[j]], bufs.at[i], gsem))
        wd = []
        for i in range(KF):
            j = g * KF + i
            gd[i].wait()
            wd.append(pltpu.async_copy(
                bufs.at[i], out.at[pl.ds((row0 + j) * ch, ch)], wsem))
        for w in wd:
            w.wait()
        return 0

    lax.fori_loop(0, n_groups, group, 0)


@functools.lru_cache(maxsize=None)
def _make_edge_gather():
    @functools.partial(
        pl.kernel,
        out_type=[jax.ShapeDtypeStruct((EP, D), F32)] * 2,
        mesh=_mesh(),
        scratch_types=[
            pltpu.VMEM((E_CPT, CH), jnp.int32),
            pltpu.VMEM((E_CPT, CH), jnp.int32),
            pltpu.VMEM((KF, CH, D), F32),
            pltpu.SemaphoreType.DMA,
            pltpu.SemaphoreType.DMA,
        ],
        compiler_params=_SC_PARAMS,
    )
    def edge_gather(tblA, s_hbm, tblB, o_hbm, gs_hbm, go_hbm,
                    sidx_v, oidx_v, bufs, gsem, wsem):
        wid = lax.axis_index("c") * NS + lax.axis_index("s")
        row0 = wid * E_CPT
        pltpu.sync_copy(s_hbm.at[pl.ds(row0, E_CPT)], sidx_v)
        pltpu.sync_copy(o_hbm.at[pl.ds(row0, E_CPT)], oidx_v)
        _pipelined_gather(tblA, sidx_v, gs_hbm, bufs, gsem, wsem,
                          row0, E_CPT, CH)
        _pipelined_gather(tblB, oidx_v, go_hbm, bufs, gsem, wsem,
                          row0, E_CPT, CH)

    return edge_gather


@functools.lru_cache(maxsize=None)
def _make_node_gather():
    @functools.partial(
        pl.kernel,
        out_type=[jax.ShapeDtypeStruct((NP, D), F32)] * 2,
        mesh=_mesh(),
        scratch_types=[
            pltpu.VMEM((N_CPT, NCH), jnp.int32),
            pltpu.VMEM((KF, NCH, D), F32),
            pltpu.SemaphoreType.DMA,
            pltpu.SemaphoreType.DMA,
        ],
        compiler_params=_SC_PARAMS,
    )
    def node_gather(tblA, tblB, idx_hbm, a_hbm, b_hbm,
                    idx_v, bufs, gsem, wsem):
        wid = lax.axis_index("c") * NS + lax.axis_index("s")
        row0 = wid * N_CPT
        pltpu.sync_copy(idx_hbm.at[pl.ds(row0, N_CPT)], idx_v)
        _pipelined_gather(tblA, idx_v, a_hbm, bufs, gsem, wsem,
                          row0, N_CPT, NCH)
        _pipelined_gather(tblB, idx_v, b_hbm, bufs, gsem, wsem,
                          row0, N_CPT, NCH)

    return node_gather


@functools.lru_cache(maxsize=None)
def _make_pred_gather():
    @functools.partial(
        pl.kernel,
        out_type=jax.ShapeDtypeStruct((EP, D), F32),
        mesh=_mesh(),
        scratch_types=[
            pltpu.VMEM((E_CPT, CH), jnp.int32),
            pltpu.VMEM((KF, CH, D), F32),
            pltpu.SemaphoreType.DMA,
            pltpu.SemaphoreType.DMA,
        ],
        compiler_params=_SC_PARAMS,
    )
    def pred_gather(tblP, p_hbm, out_hbm, idx_v, bufs, gsem, wsem):
        wid = lax.axis_index("c") * NS + lax.axis_index("s")
        row0 = wid * E_CPT
        pltpu.sync_copy(p_hbm.at[pl.ds(row0, E_CPT)], idx_v)
        _pipelined_gather(tblP, idx_v, out_hbm, bufs, gsem, wsem,
                          row0, E_CPT, CH)

    return pred_gather


def _scatter_body(v_hbm, idx_v, acc, bufs, lsem, ssem, row0,
                  cacc=None, ones_v=None):
    n_groups = E_CPT // KF

    def group(g, _):
        lds = []
        for i in range(KF):
            j = g * KF + i
            lds.append(pltpu.async_copy(
                v_hbm.at[pl.ds((row0 + j) * CH, CH)], bufs.at[i], lsem))
        scs = []
        for i in range(KF):
            j = g * KF + i
            lds[i].wait()
            scs.append(pltpu.async_copy(
                bufs.at[i], acc.at[idx_v.at[j]], ssem, add=True))
            if cacc is not None:
                scs.append(pltpu.async_copy(
                    ones_v, cacc.at[idx_v.at[j]], ssem, add=True))
        for d in scs:
            d.wait()
        return 0

    lax.fori_loop(0, n_groups, group, 0)


def _zero_dump(sid, cid, npt, zeros_hbm, acc, out_hbm, dump=False):
    if dump:
        pltpu.sync_copy(acc.at[pl.ds(sid * npt, npt)],
                        out_hbm.at[cid, pl.ds(sid * npt, npt)])
    else:
        pltpu.sync_copy(zeros_hbm.at[pl.ds(sid * npt, npt), :acc.shape[1]],
                        acc.at[pl.ds(sid * npt, npt)])


@functools.lru_cache(maxsize=None)
def _make_scatter2(with_counts):
    npt = NP // NS

    out_type = [jax.ShapeDtypeStruct((NC, NP, D), F32)]
    scratch = [
        pltpu.VMEM((E_CPT, CH), jnp.int32),
        pltpu.VMEM((E_CPT, CH), jnp.int32),
        pltpu.VMEM((KF, CH, D), F32),
        pltpu.VMEM_SHARED((NP, D), F32),
        pltpu.SemaphoreType.DMA,
        pltpu.SemaphoreType.DMA,
    ]
    if with_counts:
        out_type = out_type + [jax.ShapeDtypeStruct((NC, NP, 16), F32)]
        scratch = scratch + [pltpu.VMEM((CH, 16), F32),
                             pltpu.VMEM_SHARED((NP, 16), F32)]

    @functools.partial(
        pl.kernel,
        out_type=out_type,
        mesh=_mesh(),
        scratch_types=scratch,
        compiler_params=_SC_PARAMS,
    )
    def scatter2(*args):
        if with_counts:
            (zeros_hbm, ones_hbm, s_hbm, vs_hbm, o_hbm, vo_hbm,
             out_hbm, cnt_hbm, sidx_v, oidx_v, bufs, acc, lsem, ssem,
             ones_v, cacc) = args
        else:
            (zeros_hbm, s_hbm, vs_hbm, o_hbm, vo_hbm,
             out_hbm, sidx_v, oidx_v, bufs, acc, lsem, ssem) = args
            cacc = ones_v = None
        cid = lax.axis_index("c")
        sid = lax.axis_index("s")
        _zero_dump(sid, cid, npt, zeros_hbm, acc, out_hbm)
        if with_counts:
            _zero_dump(sid, cid, npt, zeros_hbm, cacc, cnt_hbm)
            pltpu.sync_copy(ones_hbm, ones_v)
        plsc.subcore_barrier()

        row0 = (cid * NS + sid) * E_CPT
        pltpu.sync_copy(s_hbm.at[pl.ds(row0, E_CPT)], sidx_v)
        pltpu.sync_copy(o_hbm.at[pl.ds(row0, E_CPT)], oidx_v)
        _scatter_body(vs_hbm, sidx_v, acc, bufs, lsem, ssem, row0,
                      cacc, ones_v)
        _scatter_body(vo_hbm, oidx_v, acc, bufs, lsem, ssem, row0,
                      cacc, ones_v)
        plsc.subcore_barrier()
        _zero_dump(sid, cid, npt, zeros_hbm, acc, out_hbm, dump=True)
        if with_counts:
            _zero_dump(sid, cid, npt, zeros_hbm, cacc, cnt_hbm, dump=True)

    return scatter2


def _full(shape):
    return pl.BlockSpec(shape, lambda *_: tuple(0 for _ in shape))


def _prep_body(emb_ref, pemb_ref, w1_ref, b1_ref, ta_ref, tb_ref, tp_ref):
    w1 = w1_ref[...]
    emb = emb_ref[...]
    ta_ref[...] = jnp.dot(emb, w1[0:D], preferred_element_type=F32) + b1_ref[...]
    tb_ref[...] = jnp.dot(emb, w1[2 * D:3 * D], preferred_element_type=F32)
    tp_ref[...] = jnp.dot(pemb_ref[...], w1[D:2 * D], preferred_element_type=F32)


def _prep_call(emb, pemb, w1, b1):
    return pl.pallas_call(
        _prep_body,
        out_shape=[
            jax.ShapeDtypeStruct((emb.shape[0], D), F32),
            jax.ShapeDtypeStruct((emb.shape[0], D), F32),
            jax.ShapeDtypeStruct((pemb.shape[0], D), F32),
        ],
    )(emb, pemb, w1, b1)


def _edge_body(gs_ref, go_ref, p_ref, w2_ref, b2_ref, wpn_ref,
               os_ref, oo_ref, pn_ref):
    h = jnp.maximum(gs_ref[...] + go_ref[...] + p_ref[...], 0.0)
    t = jnp.dot(h, w2_ref[...], preferred_element_type=F32) + b2_ref[...]
    t = jnp.maximum(t, 0.0)
    os_ref[...] = t[:, 0:D]
    oo_ref[...] = t[:, 2 * D:3 * D]
    pn_ref[...] = jnp.dot(t[:, D:2 * D], wpn_ref[...],
                          preferred_element_type=F32)


def _edge_body_last(gs_ref, go_ref, p_ref, w2_ref, b2_ref, os_ref, oo_ref):
    h = jnp.maximum(gs_ref[...] + go_ref[...] + p_ref[...], 0.0)
    t = jnp.dot(h, w2_ref[...], preferred_element_type=F32) + b2_ref[...]
    t = jnp.maximum(t, 0.0)
    os_ref[...] = t[:, 0:D]
    oo_ref[...] = t[:, 2 * D:3 * D]


_EB = 1024


def _edge_call(gs, go, p, w2, b2, wpn):
    grid = (EP // _EB,)
    eb = pl.BlockSpec((_EB, D), lambda i: (i, 0))
    if wpn is None:
        return pl.pallas_call(
            _edge_body_last,
            grid=grid,
            in_specs=[eb, eb, eb, _full((D, 3 * D)), _full((1, 3 * D))],
            out_specs=[eb, eb],
            out_shape=[jax.ShapeDtypeStruct((EP, D), F32)] * 2,
        )(gs, go, p, w2, b2)
    return pl.pallas_call(
        _edge_body,
        grid=grid,
        in_specs=[eb, eb, eb, _full((D, 3 * D)), _full((1, 3 * D)),
                  _full((D, D))],
        out_specs=[eb, eb, eb],
        out_shape=[jax.ShapeDtypeStruct((EP, D), F32)] * 3,
    )(gs, go, p, w2, b2, wpn)


def _node_body(p2_ref, c2_ref, w2a_ref, b2a_ref, w2b_ref, b2b_ref,
               wsn_ref, b1n_ref, won_ref, a_ref, b_ref):
    pooled = p2_ref[0] + p2_ref[1]
    cnt = jnp.maximum(c2_ref[0, :, 0:1] + c2_ref[1, :, 0:1], 1.0)
    x = pooled / cnt
    x = jnp.maximum(jnp.dot(x, w2a_ref[...], preferred_element_type=F32)
                    + b2a_ref[...], 0.0)
    obj = jnp.maximum(jnp.dot(x, w2b_ref[...], preferred_element_type=F32)
                      + b2b_ref[...], 0.0)
    a_ref[...] = jnp.dot(obj, wsn_ref[...], preferred_element_type=F32) + b1n_ref[...]
    b_ref[...] = jnp.dot(obj, won_ref[...], preferred_element_type=F32)


def _node_call(p2, c2, w2a, b2a, w2b, b2b, wsn, b1n, won):
    return pl.pallas_call(
        _node_body,
        out_shape=[jax.ShapeDtypeStruct((NP, D), F32)] * 2,
    )(p2, c2, w2a, b2a, w2b, b2b, wsn, b1n, won)


def _node_last_body(p2_ref, c2_ref, w2a_ref, b2a_ref, w2b_ref, b2b_ref,
                    wb1_ref, bb1_ref, wb2_ref, bb2_ref, out_ref):
    pooled = p2_ref[0] + p2_ref[1]
    cnt = jnp.maximum(c2_ref[0, :, 0:1] + c2_ref[1, :, 0:1], 1.0)
    x = pooled / cnt
    x = jnp.maximum(jnp.dot(x, w2a_ref[...], preferred_element_type=F32)
                    + b2a_ref[...], 0.0)
    obj = jnp.maximum(jnp.dot(x, w2b_ref[...], preferred_element_type=F32)
                      + b2b_ref[...], 0.0)
    y = jnp.maximum(jnp.dot(obj, wb1_ref[...], preferred_element_type=F32)
                    + bb1_ref[...], 0.0)
    out_ref[...] = jnp.maximum(
        jnp.dot(y, wb2_ref[...], preferred_element_type=F32) + bb2_ref[...], 0.0)


def _node_last_call(p2, c2, w2a, b2a, w2b, b2b, wb1, bb1, wb2, bb2):
    return pl.pallas_call(
        _node_last_body,
        out_shape=jax.ShapeDtypeStruct((NP, D), F32),
    )(p2, c2, w2a, b2a, w2b, b2b, wb1, bb1, wb2, bb2)


def kernel(params, objs, triples):
    s = triples[:, 0]
    p = triples[:, 1]
    o = triples[:, 2]
    pad_e = EP - N_EDGES
    s2 = jnp.concatenate([s, jnp.full((pad_e,), PAD_NODE, jnp.int32)]
                         ).reshape(E_ROWS, CH)
    o2 = jnp.concatenate([o, jnp.full((pad_e,), PAD_NODE, jnp.int32)]
                         ).reshape(E_ROWS, CH)
    p2 = jnp.concatenate([p, jnp.zeros((pad_e,), jnp.int32)]
                         ).reshape(E_ROWS, CH)
    objs2 = jnp.concatenate([objs, jnp.zeros((NP - N_NODES,), jnp.int32)]
                            ).reshape(N_ROWS, NCH)

    obj_emb = jnp.pad(params["obj_emb"], ((0, 3), (0, 0)))
    pred_emb = jnp.pad(params["pred_emb"], ((0, 2), (0, 0)))

    gconv = params["gconv"]
    w1 = [g["net1"][0] for g in gconv]
    b1 = [g["net1"][1].reshape(1, D) for g in gconv]
    w2 = [g["net1"][2] for g in gconv]
    b2 = [g["net1"][3].reshape(1, 3 * D) for g in gconv]
    w2a = [g["net2"][0] for g in gconv]
    b2a = [g["net2"][1].reshape(1, D) for g in gconv]
    w2b = [g["net2"][2] for g in gconv]
    b2b = [g["net2"][3].reshape(1, D) for g in gconv]

    box = params["box_net"]
    wb1, bb1 = box[0], box[1].reshape(1, D)
    wb2 = jnp.pad(box[2], ((0, 0), (0, D - 4)))
    bb2 = jnp.pad(box[3], ((0, D - 4),)).reshape(1, D)

    tabA, tabB, tabP = _prep_call(obj_emb, pred_emb, w1[0], b1[0])

    node_gather = _make_node_gather()
    pred_gather = _make_pred_gather()
    edge_gather = _make_edge_gather()
    scatter2c = _make_scatter2(True)
    scatter2 = _make_scatter2(False)

    A, B = node_gather(tabA, tabB, objs2)
    P = pred_gather(tabP, p2)

    zeros_np = jnp.zeros((NP, D), F32)
    ones_ch = jnp.ones((CH, 16), F32)

    for li in range(5):
        gs, go = edge_gather(A, s2, B, o2)
        if li < 4:
            os_, oo_, P = _edge_call(gs, go, P, w2[li], b2[li],
                                     w1[li + 1][D:2 * D])
        else:
            os_, oo_ = _edge_call(gs, go, P, w2[li], b2[li], None)
        if li == 0:
            pooled2, counts2 = scatter2c(zeros_np, ones_ch, s2, os_, o2, oo_)
        else:
            (pooled2,) = scatter2(zeros_np, s2, os_, o2, oo_)
        if li < 4:
            A, B = _node_call(pooled2, counts2, w2a[li], b2a[li],
                              w2b[li], b2b[li],
                              w1[li + 1][0:D], b1[li + 1],
                              w1[li + 1][2 * D:3 * D])
        else:
            boxes = _node_last_call(pooled2, counts2, w2a[li], b2a[li],
                                    w2b[li], b2b[li], wb1, bb1, wb2, bb2)

    return boxes[:N_NODES, :4]
